# baseline (speedup 1.0000x reference)
import functools
import jax
import jax.numpy as jnp
from jax import lax
from jax.experimental import pallas as pl
from jax.experimental.pallas import tpu as pltpu
from jax.experimental.pallas import tpu_sc as plsc

NR, ED, B = 6, 512, 4096
NC, NS = 2, 16
NW = NC * NS
BPW = B // NW
RC = 8
SROWS = RC * NR
WROWS = BPW * NR
NOUT = BPW // RC

_mesh = plsc.VectorSubcoreMesh(core_axis_name="c", subcore_axis_name="s")


@functools.partial(
    pl.kernel, mesh=_mesh,
    out_type=jax.ShapeDtypeStruct((B * NR, ED), jnp.float32),
    scratch_types=[
        pltpu.VMEM((SROWS,), jnp.int32),
        pltpu.VMEM((SROWS, ED), jnp.float32),
        pltpu.SemaphoreType.DMA,
        pltpu.SemaphoreType.DMA((NOUT,)),
    ],
)
def _sc_broadcast(table_hbm, idx_hbm, out_hbm, idx_v, buf_v, gsem, osems):
    wid = lax.axis_index("s") * NC + lax.axis_index("c")
    base = wid * WROWS
    pltpu.sync_copy(idx_hbm, idx_v)
    pltpu.async_copy(table_hbm.at[idx_v], buf_v, gsem).wait()
    for k in range(NOUT):
        pltpu.make_async_copy(
            buf_v, out_hbm.at[pl.ds(base + k * SROWS, SROWS)],
            osems.at[k]).start()
    for k in range(NOUT):
        pltpu.make_async_copy(
            buf_v, out_hbm.at[pl.ds(base + k * SROWS, SROWS)],
            osems.at[k]).wait()


def kernel(token_embed_weight, region_ids, batch_size):
    del batch_size
    idx_rep = jnp.tile(region_ids.astype(jnp.int32), RC)
    out2 = _sc_broadcast(token_embed_weight, idx_rep)
    return out2.reshape(B, NR, ED)


# speedup vs baseline: 1.5541x; 1.5541x over previous
import functools
import jax
import jax.numpy as jnp
from jax import lax
from jax.experimental import pallas as pl
from jax.experimental.pallas import tpu as pltpu
from jax.experimental.pallas import tpu_sc as plsc

NR, ED, B = 6, 512, 4096
NC, NS = 2, 16
NW = NC * NS
BPW = B // NW
RC = 8
SROWS = RC * NR
WROWS = BPW * NR
NOUT = BPW // RC

_mesh = plsc.VectorSubcoreMesh(core_axis_name="c", subcore_axis_name="s")


@functools.partial(
    pl.kernel, mesh=_mesh,
    out_type=jax.ShapeDtypeStruct((B, NR, ED), jnp.float32),
    scratch_types=[
        pltpu.VMEM((SROWS,), jnp.int32),
        pltpu.VMEM((SROWS, ED), jnp.float32),
        pltpu.SemaphoreType.DMA,
        pltpu.SemaphoreType.DMA((8,)),
    ],
)
def _sc_broadcast(table_hbm, idx_hbm, out_hbm, idx_v, buf_v, gsem, osems):
    wid = lax.axis_index("s") * NC + lax.axis_index("c")
    base = wid * BPW
    pltpu.sync_copy(idx_hbm, idx_v)
    pltpu.async_copy(table_hbm.at[idx_v], buf_v, gsem).wait()
    src = buf_v.at[pl.ds(0, NR)]
    for b in range(BPW):
        pltpu.make_async_copy(
            src, out_hbm.at[base + b], osems.at[b % 8]).start()
    for b in range(BPW):
        pltpu.make_async_copy(
            src, out_hbm.at[base + b], osems.at[b % 8]).wait()


def kernel(token_embed_weight, region_ids, batch_size):
    del batch_size
    idx_rep = jnp.tile(region_ids.astype(jnp.int32), RC)
    return _sc_broadcast(token_embed_weight, idx_rep)
